# baseline (speedup 1.0000x reference)
import functools

import jax
import jax.numpy as jnp
from jax import lax
from jax.experimental import pallas as pl
from jax.experimental.pallas import tpu as pltpu
from jax.experimental.pallas import tpu_sc as plsc

_L = 16
_NUM_FACTORS = 32


def _build_call(batch, num_workers, nc):
    b_per_w = batch // num_workers
    n_groups = b_per_w // _L
    mesh = plsc.VectorSubcoreMesh(core_axis_name="c", subcore_axis_name="s")

    @functools.partial(
        pl.kernel,
        out_type=jax.ShapeDtypeStruct((batch,), jnp.float32),
        mesh=mesh,
        compiler_params=pltpu.CompilerParams(
            needs_layout_passes=False, use_tc_tiling_on_sc=False),
        scratch_types=[
            pltpu.VMEM((b_per_w,), jnp.int32),
            pltpu.VMEM((b_per_w,), jnp.int32),
            pltpu.VMEM((b_per_w, _NUM_FACTORS), jnp.float32),
            pltpu.VMEM((b_per_w, _NUM_FACTORS), jnp.float32),
            pltpu.VMEM((b_per_w,), jnp.float32),
            pltpu.VMEM((b_per_w,), jnp.float32),
            pltpu.VMEM((_L,), jnp.float32),
            pltpu.VMEM((b_per_w,), jnp.float32),
            pltpu.SemaphoreType.DMA,
            pltpu.SemaphoreType.DMA,
            pltpu.SemaphoreType.DMA,
            pltpu.SemaphoreType.DMA,
        ],
    )
    def mf_kernel(uidx_hbm, pidx_hbm, uf_hbm, pf_hbm, ub_hbm, pb_hbm, g_hbm,
                  out_hbm, uidx_v, pidx_v, urow_v, prow_v, ub_v, pb_v, g_v,
                  out_v, sem_u, sem_p, sem_ub, sem_pb):
        wid = lax.axis_index("s") * nc + lax.axis_index("c")
        base = wid * b_per_w

        pltpu.sync_copy(uidx_hbm.at[pl.ds(base, b_per_w)], uidx_v)
        pltpu.sync_copy(pidx_hbm.at[pl.ds(base, b_per_w)], pidx_v)
        pltpu.sync_copy(g_hbm, g_v)

        cp_u = pltpu.async_copy(uf_hbm.at[uidx_v], urow_v, sem_u)
        cp_p = pltpu.async_copy(pf_hbm.at[pidx_v], prow_v, sem_p)
        cp_ub = pltpu.async_copy(ub_hbm.at[uidx_v], ub_v, sem_ub)
        cp_pb = pltpu.async_copy(pb_hbm.at[pidx_v], pb_v, sem_pb)
        cp_u.wait()
        cp_p.wait()
        cp_ub.wait()
        cp_pb.wait()

        lanes = lax.iota(jnp.int32, _L)
        gvec = g_v[...]

        def group_body(g, _):
            off = pl.multiple_of(g * _L, _L)
            rows = off + lanes
            acc = gvec
            for d in range(_NUM_FACTORS):
                cols = jnp.full((_L,), d, jnp.int32)
                u = plsc.load_gather(urow_v, [rows, cols])
                p = plsc.load_gather(prow_v, [rows, cols])
                acc = acc + u * p
            out_v[pl.ds(off, _L)] = acc + ub_v[pl.ds(off, _L)] + pb_v[pl.ds(off, _L)]
            return _

        lax.fori_loop(0, n_groups, group_body, None)
        pltpu.sync_copy(out_v, out_hbm.at[pl.ds(base, b_per_w)])

    return mf_kernel


def kernel(user_indices, post_indices, user_factors, post_factors,
           user_intercepts, post_intercepts, global_intercept):
    info = plsc.get_sparse_core_info()
    nc, ns = info.num_cores, info.num_subcores
    batch = user_indices.shape[0]
    call = _build_call(batch, nc * ns, nc)
    return call(
        user_indices.astype(jnp.int32),
        post_indices.astype(jnp.int32),
        user_factors,
        post_factors,
        user_intercepts.reshape(-1),
        post_intercepts.reshape(-1),
        jnp.broadcast_to(global_intercept.astype(jnp.float32), (_L,)),
    )
